# Initial kernel scaffold; baseline (speedup 1.0000x reference)
import jax
import jax.numpy as jnp
from jax.experimental import pallas as pl


def kernel(x, codebook):
    raise NotImplementedError("write your pallas kernel here")


# speedup vs baseline: 1.5204x; 1.5204x over previous
import jax
import jax.numpy as jnp
from jax import lax
from jax.experimental import pallas as pl
from jax.experimental.pallas import tpu as pltpu

_K = 1024
_C = 256
_P = 1024
_B = 8
_BETA = 0.25
_N_TOK = _B * _P
_N_ELEM = _N_TOK * _C


def _vq_kernel(x_ref, cb_ref, xq_ref, enc_ref, idx_ref, loss_ref, perp_ref,
               acc_d, acc_sq, acc_cnt):
    b = pl.program_id(0)

    @pl.when(b == 0)
    def _init():
        acc_d[...] = jnp.zeros_like(acc_d)
        acc_sq[...] = jnp.zeros_like(acc_sq)
        acc_cnt[...] = jnp.zeros_like(acc_cnt)

    xb = x_ref[0]
    cb = cb_ref[...]

    cnorm = jnp.sum(cb * cb, axis=1, keepdims=True)
    xnorm = jnp.sum(xb * xb, axis=0, keepdims=True)
    scores = lax.dot_general(cb, xb, (((1,), (0,)), ((), ())),
                             preferred_element_type=jnp.float32)
    dist = (xnorm + cnorm) - 2.0 * scores

    mval = jnp.min(dist, axis=0, keepdims=True)
    iota_k = lax.broadcasted_iota(jnp.int32, (_K, _P), 0)
    idx_row = jnp.min(jnp.where(dist == mval, iota_k, _K),
                      axis=0, keepdims=True)

    onehot_t = (iota_k == idx_row).astype(jnp.float32)

    xq = lax.dot_general(cb, onehot_t, (((0,), (0,)), ((), ())),
                         preferred_element_type=jnp.float32)

    k2 = lax.broadcasted_iota(jnp.int32, (_K, 2), 0)
    csel = lax.broadcasted_iota(jnp.int32, (_K, 2), 1)
    kcols = jnp.where(csel == 0, k2 >> 3, k2 & 7).astype(jnp.float32)
    parts = lax.dot_general(onehot_t, kcols, (((0,), (0,)), ((), ())),
                            preferred_element_type=jnp.float32)
    idx_col = (parts[:, 0:1] * 8.0 + parts[:, 1:2]).astype(jnp.int32)

    iota_lane = lax.broadcasted_iota(jnp.int32, (_P, _K), 1)
    onehot_p = (iota_lane == idx_col).astype(jnp.float32)

    enc_ref[...] = onehot_p
    idx_ref[...] = idx_col
    xq_ref[0] = xb + (xq - xb)

    diff = xb - xq
    acc_d[...] += jnp.sum(diff, axis=0, keepdims=True)
    acc_sq[...] += jnp.sum(diff * diff, axis=0, keepdims=True)
    acc_cnt[...] += jnp.sum(onehot_p, axis=0, keepdims=True)

    @pl.when(b == _B - 1)
    def _fin():
        inv_n = 1.0 / _N_ELEM
        sum_d = jnp.sum(acc_d[...], keepdims=True)
        sum_sq = jnp.sum(acc_sq[...], keepdims=True)
        loss_ref[...] = _BETA * sum_d * inv_n + sum_sq * inv_n
        e_mean = acc_cnt[...] * (1.0 / _N_TOK)
        ent = jnp.sum(e_mean * jnp.log(e_mean + 1e-10), keepdims=True)
        perp_ref[...] = jnp.exp(-ent)


@jax.jit
def kernel(x, codebook):
    x3 = x.reshape(_B, _C, _P)
    out_shapes = (
        jax.ShapeDtypeStruct((_B, _C, _P), jnp.float32),
        jax.ShapeDtypeStruct((_N_TOK, _K), jnp.float32),
        jax.ShapeDtypeStruct((_N_TOK, 1), jnp.int32),
        jax.ShapeDtypeStruct((1, 1), jnp.float32),
        jax.ShapeDtypeStruct((1, 1), jnp.float32),
    )
    xq, enc, idx, loss, perp = pl.pallas_call(
        _vq_kernel,
        grid=(_B,),
        in_specs=[
            pl.BlockSpec((1, _C, _P), lambda b: (b, 0, 0)),
            pl.BlockSpec((_K, _C), lambda b: (0, 0)),
        ],
        out_specs=(
            pl.BlockSpec((1, _C, _P), lambda b: (b, 0, 0)),
            pl.BlockSpec((_P, _K), lambda b: (b, 0)),
            pl.BlockSpec((_P, 1), lambda b: (b, 0)),
            pl.BlockSpec((1, 1), lambda b: (0, 0)),
            pl.BlockSpec((1, 1), lambda b: (0, 0)),
        ),
        out_shape=out_shapes,
        scratch_shapes=[
            pltpu.VMEM((1, _P), jnp.float32),
            pltpu.VMEM((1, _P), jnp.float32),
            pltpu.VMEM((1, _K), jnp.float32),
        ],
    )(x3, codebook)
    xq4 = xq.reshape(_B, _C, 32, 32)
    return (xq4, loss[0, 0], perp[0, 0], enc, idx)
